# baseline (speedup 1.0000x reference)
import jax
import jax.numpy as jnp
from jax.experimental import pallas as pl
from jax.experimental.pallas import tpu as pltpu

N = 4096
F = 512
H = 16
C = 16
B = 1024
I = N // B
P = I * (I + 1) // 2
XB = 512
XCH = N // XB


def _pair(k):
    bi = jnp.int32(0)
    for t in range(1, I):
        bi = bi + (k >= t * I - t * (t - 1) // 2).astype(jnp.int32)
    bj = k - (bi * I - bi * (bi - 1) // 2) + bi
    return bi, bj


def _clip_block(ap_ref, masked):
    u = jnp.clip(ap_ref[...], 0.0, 1.0)
    if masked:
        r = jax.lax.broadcasted_iota(jnp.int32, (B, B), 0)
        c = jax.lax.broadcasted_iota(jnp.int32, (B, B), 1)
        u = jnp.where(r < c, u, 0.0)
    return u


def _deg_body(x_ref, w1_ref, ap_ref, dis_ref, xw1_ref, acc, accc):
    k = pl.program_id(0)
    bi, bj = _pair(k)

    @pl.when(k == 0)
    def _init():
        acc[...] = jnp.zeros_like(acc)
        accc[...] = jnp.zeros_like(accc)

    def contrib(masked):
        T = _clip_block(ap_ref, masked)
        acc[0, pl.ds(bj * B, B)] += jnp.sum(T, axis=0)
        accc[pl.ds(bi * B, B), :] += jnp.sum(T, axis=1, keepdims=True)

    @pl.when(bi == bj)
    def _diag():
        contrib(True)

    @pl.when(bi != bj)
    def _off():
        contrib(False)

    @pl.when(k < XCH)
    def _xw1():
        xw1_ref[pl.ds(k * XB, XB), :] = jnp.dot(
            x_ref[...], w1_ref[...], preferred_element_type=jnp.float32)

    @pl.when(k == P - 1)
    def _fini():
        dis_ref[...] = jax.lax.rsqrt(acc[...] + accc[...].T + 1.0)


def _layer_body(first_mm, last_mm):
    def body(dis_ref, dense_refs, ap_ref, out_ref, u_s, uT_s, acc_s, accT_s):
        k = pl.program_id(0)
        bi, bj = _pair(k)

        @pl.when(k == 0)
        def _init():
            u = dis_ref[...] * first_mm(dense_refs)
            u_s[...] = u
            uT_s[...] = u.astype(jnp.bfloat16).T
            acc_s[...] = jnp.zeros_like(acc_s)
            accT_s[...] = jnp.zeros_like(accT_s)

        def contrib(masked):
            T = _clip_block(ap_ref, masked).astype(jnp.bfloat16)
            vj = u_s[pl.ds(bj * B, B), :].astype(jnp.bfloat16)
            viT = uT_s[:, pl.ds(bi * B, B)]
            acc_s[pl.ds(bi * B, B), :] += jnp.dot(
                T, vj, preferred_element_type=jnp.float32)
            accT_s[:, pl.ds(bj * B, B)] += jnp.dot(
                viT, T, preferred_element_type=jnp.float32)

        @pl.when(bi == bj)
        def _diag():
            contrib(True)

        @pl.when(bi != bj)
        def _off():
            contrib(False)

        @pl.when(k == P - 1)
        def _fini():
            tot = acc_s[...] + accT_s[...].T + u_s[...]
            out_ref[...] = last_mm(dense_refs, dis_ref[...] * tot)

    return body


def _full(shape):
    return pl.BlockSpec(shape, lambda k: (0,) * len(shape))


def _ap_spec():
    return pl.BlockSpec((B, B), _pair)


def kernel(x, A_param, W1, b1, W2, b2):
    assert x.shape == (N, F) and A_param.shape == (N, N)
    b1r = b1.reshape(1, H)
    b2r = b2.reshape(1, C)

    grid = (P,)

    dis_row, xw1 = pl.pallas_call(
        _deg_body,
        grid=grid,
        in_specs=[
            pl.BlockSpec((XB, F), lambda k: (jnp.minimum(k, XCH - 1), 0)),
            _full((F, H)),
            _ap_spec(),
        ],
        out_specs=[_full((1, N)), _full((N, H))],
        out_shape=[
            jax.ShapeDtypeStruct((1, N), jnp.float32),
            jax.ShapeDtypeStruct((N, H), jnp.float32),
        ],
        scratch_shapes=[
            pltpu.VMEM((1, N), jnp.float32),
            pltpu.VMEM((N, 1), jnp.float32),
        ],
    )(x, W1, A_param)
    dis = dis_row.reshape(N, 1)

    def l1_first(refs):
        xw1_ref, b1_ref, w2_ref = refs
        return xw1_ref[...]

    def l1_last(refs, y):
        xw1_ref, b1_ref, w2_ref = refs
        h = jax.nn.relu(y + b1_ref[...])
        return jnp.dot(h, w2_ref[...], preferred_element_type=jnp.float32)

    def body1(dis_ref, xw1_ref, b1_ref, w2_ref, ap_ref, out_ref,
              u_s, uT_s, acc_s, accT_s):
        _layer_body(l1_first, l1_last)(
            dis_ref, (xw1_ref, b1_ref, w2_ref), ap_ref, out_ref,
            u_s, uT_s, acc_s, accT_s)

    v2 = pl.pallas_call(
        body1,
        grid=grid,
        in_specs=[_full((N, 1)), _full((N, H)),
                  _full((1, H)), _full((H, C)), _ap_spec()],
        out_specs=_full((N, C)),
        out_shape=jax.ShapeDtypeStruct((N, C), jnp.float32),
        scratch_shapes=[
            pltpu.VMEM((N, H), jnp.float32),
            pltpu.VMEM((H, N), jnp.bfloat16),
            pltpu.VMEM((N, H), jnp.float32),
            pltpu.VMEM((H, N), jnp.float32),
        ],
    )(dis, xw1, b1r, W2, A_param)

    def l2_first(refs):
        (v2_ref, b2_ref) = refs
        return v2_ref[...]

    def l2_last(refs, y):
        (v2_ref, b2_ref) = refs
        return y + b2_ref[...]

    def body2(dis_ref, v2_ref, b2_ref, ap_ref, out_ref,
              u_s, uT_s, acc_s, accT_s):
        _layer_body(l2_first, l2_last)(
            dis_ref, (v2_ref, b2_ref), ap_ref, out_ref,
            u_s, uT_s, acc_s, accT_s)

    out = pl.pallas_call(
        body2,
        grid=grid,
        in_specs=[_full((N, 1)), _full((N, C)), _full((1, C)), _ap_spec()],
        out_specs=_full((N, C)),
        out_shape=jax.ShapeDtypeStruct((N, C), jnp.float32),
        scratch_shapes=[
            pltpu.VMEM((N, C), jnp.float32),
            pltpu.VMEM((C, N), jnp.bfloat16),
            pltpu.VMEM((N, C), jnp.float32),
            pltpu.VMEM((C, N), jnp.float32),
        ],
    )(dis, v2, b2r, A_param)

    return out


# speedup vs baseline: 1.1395x; 1.1395x over previous
import jax
import jax.numpy as jnp
from jax.experimental import pallas as pl
from jax.experimental.pallas import tpu as pltpu

N = 4096
F = 512
H = 16
C = 16
B = 1024
I = N // B
P = I * (I + 1) // 2
XB = 512
XCH = N // XB


def _pair(k):
    bi = jnp.int32(0)
    for t in range(1, I):
        bi = bi + (k >= t * I - t * (t - 1) // 2).astype(jnp.int32)
    bj = k - (bi * I - bi * (bi - 1) // 2) + bi
    return bi, bj


def _clip_block(ap_ref, masked):
    u = jnp.clip(ap_ref[...], 0.0, 1.0)
    if masked:
        r = jax.lax.broadcasted_iota(jnp.int32, (B, B), 0)
        c = jax.lax.broadcasted_iota(jnp.int32, (B, B), 1)
        u = jnp.where(r < c, u, 0.0)
    return u


def _body(x_ref, w1_ref, b1_ref, w2_ref, b2_ref, ap_ref, out_ref,
          accr, accc, dis_s, u_s, uT_s, acc_s, accT_s, vnext_s):
    p, k = pl.program_id(0), pl.program_id(1)
    bi, bj = _pair(k)

    @pl.when((p == 0) & (k == 0))
    def _init0():
        accr[...] = jnp.zeros_like(accr)
        accc[...] = jnp.zeros_like(accc)

    def degsum(masked):
        T = _clip_block(ap_ref, masked)
        accr[0, pl.ds(bj * B, B)] += jnp.sum(T, axis=0)
        accc[pl.ds(bi * B, B), :] += jnp.sum(T, axis=1, keepdims=True)

    @pl.when((p == 0) & (bi == bj))
    def _deg_diag():
        degsum(True)

    @pl.when((p == 0) & (bi != bj))
    def _deg_off():
        degsum(False)

    @pl.when((p == 0) & (k < XCH))
    def _xw1():
        vnext_s[pl.ds(k * XB, XB), :] = jnp.dot(
            x_ref[...], w1_ref[...], preferred_element_type=jnp.float32)

    @pl.when((p == 0) & (k == P - 1))
    def _fini0():
        dis_s[...] = jax.lax.rsqrt(accc[...] + accr[...].T + 1.0)

    @pl.when((p > 0) & (k == 0))
    def _initp():
        u = dis_s[...] * vnext_s[...]
        u_s[...] = u
        uT_s[...] = u.astype(jnp.bfloat16).T
        acc_s[...] = jnp.zeros_like(acc_s)
        accT_s[...] = jnp.zeros_like(accT_s)

    def prop(masked):
        T = _clip_block(ap_ref, masked).astype(jnp.bfloat16)
        vj = u_s[pl.ds(bj * B, B), :].astype(jnp.bfloat16)
        viT = uT_s[:, pl.ds(bi * B, B)]
        acc_s[pl.ds(bi * B, B), :] += jnp.dot(
            T, vj, preferred_element_type=jnp.float32)
        accT_s[:, pl.ds(bj * B, B)] += jnp.dot(
            viT, T, preferred_element_type=jnp.float32)

    @pl.when((p > 0) & (bi == bj))
    def _prop_diag():
        prop(True)

    @pl.when((p > 0) & (bi != bj))
    def _prop_off():
        prop(False)

    @pl.when((p == 1) & (k == P - 1))
    def _fini1():
        y = dis_s[...] * (acc_s[...] + accT_s[...].T + u_s[...])
        h = jax.nn.relu(y + b1_ref[...])
        vnext_s[...] = jnp.dot(h, w2_ref[...],
                               preferred_element_type=jnp.float32)

    @pl.when((p == 2) & (k == P - 1))
    def _fini2():
        y = dis_s[...] * (acc_s[...] + accT_s[...].T + u_s[...])
        out_ref[...] = y + b2_ref[...]


def kernel(x, A_param, W1, b1, W2, b2):
    assert x.shape == (N, F) and A_param.shape == (N, N)
    assert H == C
    b1r = b1.reshape(1, H)
    b2r = b2.reshape(1, C)

    def full(shape):
        return pl.BlockSpec(shape, lambda p, k: (0,) * len(shape))

    out = pl.pallas_call(
        _body,
        grid=(3, P),
        in_specs=[
            pl.BlockSpec((XB, F), lambda p, k: (jnp.minimum(k, XCH - 1), 0)),
            full((F, H)),
            full((1, H)),
            full((H, C)),
            full((1, C)),
            pl.BlockSpec((B, B), lambda p, k: _pair(k)),
        ],
        out_specs=full((N, C)),
        out_shape=jax.ShapeDtypeStruct((N, C), jnp.float32),
        scratch_shapes=[
            pltpu.VMEM((1, N), jnp.float32),
            pltpu.VMEM((N, 1), jnp.float32),
            pltpu.VMEM((N, 1), jnp.float32),
            pltpu.VMEM((N, H), jnp.float32),
            pltpu.VMEM((H, N), jnp.bfloat16),
            pltpu.VMEM((N, H), jnp.float32),
            pltpu.VMEM((H, N), jnp.float32),
            pltpu.VMEM((N, H), jnp.float32),
        ],
    )(x, W1, b1r, W2, b2r, A_param)

    return out


# speedup vs baseline: 1.2435x; 1.0913x over previous
import jax
import jax.numpy as jnp
from jax.experimental import pallas as pl
from jax.experimental.pallas import tpu as pltpu

N = 4096
F = 512
H = 16
C = 16
B = 1024
I = N // B
P = I * (I + 1) // 2
XB = 512
XCH = N // XB


def _pair(k):
    bi = jnp.int32(0)
    for t in range(1, I):
        bi = bi + (k >= t * I - t * (t - 1) // 2).astype(jnp.int32)
    bj = k - (bi * I - bi * (bi - 1) // 2) + bi
    return bi, bj


def _clip_block(ap_ref, masked):
    u = jnp.clip(ap_ref[...], 0.0, 1.0)
    if masked:
        r = jax.lax.broadcasted_iota(jnp.int32, (B, B), 0)
        c = jax.lax.broadcasted_iota(jnp.int32, (B, B), 1)
        u = jnp.where(r < c, u, 0.0)
    return u


def _body(x0_ref, x1_ref, w1_ref, b1_ref, w2_ref, b2_ref,
          ap0_ref, ap1_ref, out_ref,
          accr, accc, dis_s, u_s, uT_s, acc_s, accT_s, vnext_s):
    p, s = pl.program_id(0), pl.program_id(1)

    @pl.when((p == 0) & (s == 0))
    def _init0():
        accr[...] = jnp.zeros_like(accr)
        accc[...] = jnp.zeros_like(accc)

    def degsum(ap_ref, bi, bj, masked):
        T = _clip_block(ap_ref, masked)
        accr[0, pl.ds(bj * B, B)] += jnp.sum(T, axis=0)
        accc[pl.ds(bi * B, B), :] += jnp.sum(T, axis=1, keepdims=True)

    def prop(ap_ref, bi, bj, masked):
        T = _clip_block(ap_ref, masked).astype(jnp.bfloat16)
        vj = u_s[pl.ds(bj * B, B), :].astype(jnp.bfloat16)
        viT = uT_s[:, pl.ds(bi * B, B)]
        acc_s[pl.ds(bi * B, B), :] += jnp.dot(
            T, vj, preferred_element_type=jnp.float32)
        accT_s[:, pl.ds(bj * B, B)] += jnp.dot(
            viT, T, preferred_element_type=jnp.float32)

    @pl.when((p == 0) & (s * 2 < XCH))
    def _xw1a():
        vnext_s[pl.ds(s * 2 * XB, XB), :] = jnp.dot(
            x0_ref[...], w1_ref[...], preferred_element_type=jnp.float32)

    @pl.when((p == 0) & (s * 2 + 1 < XCH))
    def _xw1b():
        vnext_s[pl.ds((s * 2 + 1) * XB, XB), :] = jnp.dot(
            x1_ref[...], w1_ref[...], preferred_element_type=jnp.float32)

    @pl.when((p > 0) & (s == 0))
    def _initp():
        u = dis_s[...] * vnext_s[...]
        u_s[...] = u
        uT_s[...] = u.astype(jnp.bfloat16).T
        acc_s[...] = jnp.zeros_like(acc_s)
        accT_s[...] = jnp.zeros_like(accT_s)

    for half, ap_ref in ((0, ap0_ref), (1, ap1_ref)):
        k = s * 2 + half
        bi, bj = _pair(k)

        @pl.when((p == 0) & (bi == bj))
        def _deg_diag(ap_ref=ap_ref, bi=bi, bj=bj):
            degsum(ap_ref, bi, bj, True)

        @pl.when((p == 0) & (bi != bj))
        def _deg_off(ap_ref=ap_ref, bi=bi, bj=bj):
            degsum(ap_ref, bi, bj, False)

        @pl.when((p > 0) & (bi == bj))
        def _prop_diag(ap_ref=ap_ref, bi=bi, bj=bj):
            prop(ap_ref, bi, bj, True)

        @pl.when((p > 0) & (bi != bj))
        def _prop_off(ap_ref=ap_ref, bi=bi, bj=bj):
            prop(ap_ref, bi, bj, False)

    @pl.when((p == 0) & (s == P // 2 - 1))
    def _fini0():
        dis_s[...] = jax.lax.rsqrt(accc[...] + accr[...].T + 1.0)

    @pl.when((p == 1) & (s == P // 2 - 1))
    def _fini1():
        y = dis_s[...] * (acc_s[...] + accT_s[...].T + u_s[...])
        h = jax.nn.relu(y + b1_ref[...])
        vnext_s[...] = jnp.dot(h, w2_ref[...],
                               preferred_element_type=jnp.float32)

    @pl.when((p == 2) & (s == P // 2 - 1))
    def _fini2():
        y = dis_s[...] * (acc_s[...] + accT_s[...].T + u_s[...])
        out_ref[...] = y + b2_ref[...]


def kernel(x, A_param, W1, b1, W2, b2):
    assert x.shape == (N, F) and A_param.shape == (N, N)
    assert H == C
    b1r = b1.reshape(1, H)
    b2r = b2.reshape(1, C)

    def full(shape):
        return pl.BlockSpec(shape, lambda p, k: (0,) * len(shape))

    out = pl.pallas_call(
        _body,
        grid=(3, P // 2),
        in_specs=[
            pl.BlockSpec((XB, F),
                         lambda p, s: (jnp.minimum(s * 2, XCH - 1), 0)),
            pl.BlockSpec((XB, F),
                         lambda p, s: (jnp.minimum(s * 2 + 1, XCH - 1), 0)),
            full((F, H)),
            full((1, H)),
            full((H, C)),
            full((1, C)),
            pl.BlockSpec((B, B), lambda p, s: _pair(s * 2)),
            pl.BlockSpec((B, B), lambda p, s: _pair(s * 2 + 1)),
        ],
        out_specs=full((N, C)),
        out_shape=jax.ShapeDtypeStruct((N, C), jnp.float32),
        scratch_shapes=[
            pltpu.VMEM((1, N), jnp.float32),
            pltpu.VMEM((N, 1), jnp.float32),
            pltpu.VMEM((N, 1), jnp.float32),
            pltpu.VMEM((N, H), jnp.float32),
            pltpu.VMEM((H, N), jnp.bfloat16),
            pltpu.VMEM((N, H), jnp.float32),
            pltpu.VMEM((H, N), jnp.float32),
            pltpu.VMEM((N, H), jnp.float32),
        ],
    )(x, x, W1, b1r, W2, b2r, A_param, A_param)

    return out
